# Initial kernel scaffold; baseline (speedup 1.0000x reference)
import jax
import jax.numpy as jnp
from jax.experimental import pallas as pl


def kernel(x, edge_index, edge_attr, W1, a_src1, a_dst1, b1, W2, a_src2, a_dst2, b2, Wl, bl):
    raise NotImplementedError("write your pallas kernel here")


# speedup vs baseline: 24.3371x; 24.3371x over previous
import dataclasses
import functools

import jax
import jax.numpy as jnp
from jax import lax
from jax.experimental import pallas as pl
from jax.experimental.pallas import tpu as pltpu
from jax.experimental.pallas import tpu_sc as plsc

N = 10000
D = 128
NC = 2
NS = 16
NW = NC * NS
EB = 128
NG = 9
GB = 9
NBLK = NG * GB
CH = NBLK * EB
EPAD = NW * CH
NPAD = 10240
RPT = NPAD // NS
BN = 1024
GRID = NPAD // BN

_sc_params = pltpu.CompilerParams()
if "needs_layout_passes" in pltpu.CompilerParams.__dataclass_fields__:
    _sc_params = dataclasses.replace(_sc_params, needs_layout_passes=False)

_vector_mesh = plsc.VectorSubcoreMesh(core_axis_name="c", subcore_axis_name="s")


def _tc_embed_body(x_ref, W_ref, as_ref, ad_ref, h_ref, s_ref, d_ref, m_ref):
    h = jnp.dot(x_ref[...], W_ref[...], preferred_element_type=jnp.float32)
    h_ref[...] = h
    s = jnp.dot(h, as_ref[...], preferred_element_type=jnp.float32)
    d = jnp.dot(h, ad_ref[...], preferred_element_type=jnp.float32)
    s_ref[...] = s
    d_ref[...] = d
    sd = s + d
    m_ref[...] = jnp.where(sd > 0.0, sd, 0.2 * sd)


def _tc_embed(x, W, a_s, a_d):
    return pl.pallas_call(
        _tc_embed_body,
        grid=(GRID,),
        in_specs=[
            pl.BlockSpec((BN, D), lambda i: (i, 0)),
            pl.BlockSpec((D, D), lambda i: (0, 0)),
            pl.BlockSpec((D, 1), lambda i: (0, 0)),
            pl.BlockSpec((D, 1), lambda i: (0, 0)),
        ],
        out_specs=[
            pl.BlockSpec((BN, D), lambda i: (i, 0)),
            pl.BlockSpec((BN, 1), lambda i: (i, 0)),
            pl.BlockSpec((BN, 1), lambda i: (i, 0)),
            pl.BlockSpec((BN, 1), lambda i: (i, 0)),
        ],
        out_shape=[
            jax.ShapeDtypeStruct((NPAD, D), jnp.float32),
            jax.ShapeDtypeStruct((NPAD, 1), jnp.float32),
            jax.ShapeDtypeStruct((NPAD, 1), jnp.float32),
            jax.ShapeDtypeStruct((NPAD, 1), jnp.float32),
        ],
    )(x, W, a_s, a_d)


def _normalize(p_ref, den_ref, b_ref):
    i = pl.program_id(0)
    den = den_ref[:, pl.ds(i * BN, BN)]
    dsum = jnp.sum(den, axis=0)
    inv = 1.0 / (dsum + 1e-16)
    agg = (p_ref[0] + p_ref[1]) * inv[:, None] + b_ref[...]
    return jnp.maximum(agg, 0.0)


def _tc_merge_body(p_ref, den_ref, b_ref, W_ref, as_ref, ad_ref,
                   h_ref, s_ref, d_ref, m_ref):
    g = _normalize(p_ref, den_ref, b_ref)
    h = jnp.dot(g, W_ref[...], preferred_element_type=jnp.float32)
    h_ref[...] = h
    s = jnp.dot(h, as_ref[...], preferred_element_type=jnp.float32)
    d = jnp.dot(h, ad_ref[...], preferred_element_type=jnp.float32)
    s_ref[...] = s
    d_ref[...] = d
    sd = s + d
    m_ref[...] = jnp.where(sd > 0.0, sd, 0.2 * sd)


def _tc_merge(p, den, b, W, a_s, a_d):
    return pl.pallas_call(
        _tc_merge_body,
        grid=(GRID,),
        in_specs=[
            pl.BlockSpec((NC, BN, D), lambda i: (0, i, 0)),
            pl.BlockSpec((NW, NPAD), lambda i: (0, 0)),
            pl.BlockSpec((1, D), lambda i: (0, 0)),
            pl.BlockSpec((D, D), lambda i: (0, 0)),
            pl.BlockSpec((D, 1), lambda i: (0, 0)),
            pl.BlockSpec((D, 1), lambda i: (0, 0)),
        ],
        out_specs=[
            pl.BlockSpec((BN, D), lambda i: (i, 0)),
            pl.BlockSpec((BN, 1), lambda i: (i, 0)),
            pl.BlockSpec((BN, 1), lambda i: (i, 0)),
            pl.BlockSpec((BN, 1), lambda i: (i, 0)),
        ],
        out_shape=[
            jax.ShapeDtypeStruct((NPAD, D), jnp.float32),
            jax.ShapeDtypeStruct((NPAD, 1), jnp.float32),
            jax.ShapeDtypeStruct((NPAD, 1), jnp.float32),
            jax.ShapeDtypeStruct((NPAD, 1), jnp.float32),
        ],
    )(p, den, b, W, a_s, a_d)


def _tc_final_body(p_ref, den_ref, b_ref, Wl_ref, bl_ref, y_ref):
    g = _normalize(p_ref, den_ref, b_ref)
    y_ref[...] = (jnp.dot(g, Wl_ref[...], preferred_element_type=jnp.float32)
                  + bl_ref[...])


def _tc_final(p, den, b, Wl, bl):
    return pl.pallas_call(
        _tc_final_body,
        grid=(GRID,),
        in_specs=[
            pl.BlockSpec((NC, BN, D), lambda i: (0, i, 0)),
            pl.BlockSpec((NW, NPAD), lambda i: (0, 0)),
            pl.BlockSpec((1, D), lambda i: (0, 0)),
            pl.BlockSpec((D, 1), lambda i: (0, 0)),
            pl.BlockSpec((1, 1), lambda i: (0, 0)),
        ],
        out_specs=pl.BlockSpec((BN, 1), lambda i: (i, 0)),
        out_shape=jax.ShapeDtypeStruct((NPAD, 1), jnp.float32),
    )(p, den, b, Wl, bl)


def _sc_weights_body(e_true, s_hbm, d_hbm, m_hbm, src_hbm, dst_hbm,
                     w_hbm, denp_hbm,
                     s_v, d_v, m_v, den_v, src_v, dst_v, w_v):
    cid = lax.axis_index("c")
    sid = lax.axis_index("s")
    wid = sid * NC + cid

    pltpu.sync_copy(s_hbm, s_v)
    pltpu.sync_copy(d_hbm, d_v)
    pltpu.sync_copy(m_hbm, m_v)
    base = pl.multiple_of(wid * CH, 8)
    pltpu.sync_copy(src_hbm.at[pl.ds(base, CH)], src_v)
    pltpu.sync_copy(dst_hbm.at[pl.ds(base, CH)], dst_v)

    zeros16 = jnp.zeros((16,), jnp.float32)

    @pl.loop(0, NPAD // 16)
    def _zero_den(i):
        den_v[pl.ds(i * 16, 16)] = zeros16

    @pl.loop(0, NBLK)
    def _block(j):
        for c in range(8):
            off = j * EB + c * 16
            sidx = src_v[pl.ds(off, 16)]
            didx = dst_v[pl.ds(off, 16)]
            sg = plsc.load_gather(s_v, [sidx])
            dg = plsc.load_gather(d_v, [didx])
            mg = plsc.load_gather(m_v, [didx])
            e = sg + dg
            e = jnp.where(e > 0.0, e, 0.2 * e)
            w = jnp.exp(jnp.minimum(e - mg, 80.0))
            gid = base + off + lax.iota(jnp.int32, 16)
            w = jnp.where(gid < e_true, w, 0.0)
            w_v[pl.ds(off, 16)] = w
            plsc.addupdate_scatter(den_v, [didx], w)

    pltpu.sync_copy(w_v, w_hbm.at[pl.ds(base, CH)])
    pltpu.sync_copy(den_v,
                    denp_hbm.at[pl.ds(pl.multiple_of(wid * NPAD, 8), NPAD)])


def _sc_rows_body(h_hbm, src_hbm, dst_hbm, w_hbm, outp_hbm,
                  src_v, dst_v, w_v, rows_v, acc_sh):
    cid = lax.axis_index("c")
    sid = lax.axis_index("s")
    wid = sid * NC + cid

    zeros16 = jnp.zeros((16,), jnp.float32)

    @pl.loop(0, EB)
    def _zero_rows(r):
        for c in range(8):
            rows_v[r, pl.ds(c * 16, 16)] = zeros16

    @pl.loop(0, RPT // EB)
    def _zero_acc(k):
        pltpu.sync_copy(rows_v,
                        acc_sh.at[pl.ds(pl.multiple_of(sid * RPT + k * EB, 8),
                                        EB)])

    plsc.subcore_barrier()

    @pl.loop(0, NG)
    def _group(g):
        pltpu.sync_copy(src_hbm.at[wid, g], src_v)
        pltpu.sync_copy(dst_hbm.at[wid, g], dst_v)
        pltpu.sync_copy(w_hbm.at[wid, g], w_v)

        for jj in range(GB):
            pltpu.sync_copy(h_hbm.at[src_v.at[jj]], rows_v)

            @pl.loop(0, EB)
            def _scale(r):
                wb = plsc.load_gather(
                    w_v, [jnp.full((16,), jj, jnp.int32),
                          jnp.full((16,), r, jnp.int32)])
                for c in range(8):
                    rows_v[r, pl.ds(c * 16, 16)] = (
                        rows_v[r, pl.ds(c * 16, 16)] * wb)

            pltpu.sync_copy(rows_v, acc_sh.at[dst_v.at[jj]], add=True)

    plsc.subcore_barrier()

    row0 = pl.multiple_of(sid * RPT, 8)
    pltpu.sync_copy(acc_sh.at[pl.ds(row0, RPT)],
                    outp_hbm.at[cid, pl.ds(row0, RPT)])


def _sc_gat(e_true, h, s, d, m, src, dst):
    wkern = pl.kernel(
        functools.partial(_sc_weights_body, e_true),
        out_type=[
            jax.ShapeDtypeStruct((EPAD,), jnp.float32),
            jax.ShapeDtypeStruct((NW * NPAD,), jnp.float32),
        ],
        mesh=_vector_mesh,
        scratch_types=[
            pltpu.VMEM((NPAD,), jnp.float32),
            pltpu.VMEM((NPAD,), jnp.float32),
            pltpu.VMEM((NPAD,), jnp.float32),
            pltpu.VMEM((NPAD,), jnp.float32),
            pltpu.VMEM((CH,), jnp.int32),
            pltpu.VMEM((CH,), jnp.int32),
            pltpu.VMEM((CH,), jnp.float32),
        ],
        compiler_params=_sc_params,
    )
    w, denp = wkern(s, d, m, src.reshape(-1), dst.reshape(-1))
    w = w.reshape(NW, NG, GB, EB)

    rkern = pl.kernel(
        _sc_rows_body,
        out_type=jax.ShapeDtypeStruct((NC, NPAD, D), jnp.float32),
        mesh=_vector_mesh,
        scratch_types=[
            pltpu.VMEM((GB, EB), jnp.int32),
            pltpu.VMEM((GB, EB), jnp.int32),
            pltpu.VMEM((GB, EB), jnp.float32),
            pltpu.VMEM((EB, D), jnp.float32),
            pltpu.VMEM_SHARED((NPAD, D), jnp.float32),
        ],
        compiler_params=_sc_params,
    )
    outp = rkern(h, src, dst, w)
    return outp, denp


def kernel(x, edge_index, edge_attr, W1, a_src1, a_dst1, b1,
           W2, a_src2, a_dst2, b2, Wl, bl):
    del edge_attr
    n = x.shape[0]
    e_true = edge_index.shape[1] + n
    loop = jnp.arange(n, dtype=edge_index.dtype)
    pad = jnp.zeros((EPAD - e_true,), edge_index.dtype)
    src = jnp.concatenate([edge_index[0], loop, pad]).reshape(NW, NG, GB, EB)
    dst = jnp.concatenate([edge_index[1], loop, pad]).reshape(NW, NG, GB, EB)

    a_s1 = a_src1.reshape(D, 1)
    a_d1 = a_dst1.reshape(D, 1)
    a_s2 = a_src2.reshape(D, 1)
    a_d2 = a_dst2.reshape(D, 1)

    x_pad = jnp.concatenate([x, jnp.zeros((NPAD - N, D), x.dtype)])
    h1, s1, d1, m1 = _tc_embed(x_pad, W1, a_s1, a_d1)
    p1, den1 = _sc_gat(e_true, h1, s1.reshape(-1), d1.reshape(-1),
                       m1.reshape(-1), src, dst)
    h2, s2, d2, m2 = _tc_merge(p1, den1.reshape(NW, NPAD), b1.reshape(1, D),
                               W2, a_s2, a_d2)
    p2, den2 = _sc_gat(e_true, h2, s2.reshape(-1), d2.reshape(-1),
                       m2.reshape(-1), src, dst)
    y = _tc_final(p2, den2.reshape(NW, NPAD), b2.reshape(1, D),
                  Wl, bl.reshape(1, 1))
    return y[:N]
